# baseline (speedup 1.0000x reference)
import functools
import jax
import jax.numpy as jnp
import numpy as np
from jax.experimental import pallas as pl
from jax.experimental.pallas import tpu as pltpu
from jax._src.pallas.mosaic import sc_core as plsc

_B, _S, _D_IN, _D_OUT = 2, 2048, 2048, 2048
_E, _R = 8, 64
_RMOE = _E * _R
_SCALING = 16.0 / 64.0
_T = _B * _S

_BM = 512
_NBLK = _T // _BM
_TOK_PER_SUB = _T // 32
_GROUPS = _TOK_PER_SUB // 16


def _bodyA(x_ref, wg_ref, wa_ref, lt_ref, xb_ref, h_ref, wa16_ref):
    @pl.when(pl.program_id(0) == 0)
    def _():
        wa16_ref[...] = wa_ref[...].astype(jnp.bfloat16)

    xb = x_ref[...].astype(jnp.bfloat16)
    xb_ref[...] = xb
    lt_ref[...] = jax.lax.dot_general(
        wg_ref[...].astype(jnp.bfloat16), xb, (((1,), (1,)), ((), ())),
        preferred_element_type=jnp.float32)
    h = jax.lax.dot_general(
        xb, wa16_ref[...], (((1,), (1,)), ((), ())),
        preferred_element_type=jnp.float32)
    h_ref[...] = h.astype(jnp.bfloat16)


def _sc_gate_body(logits_ref, out_ref, vin, vout, sem_in, sem_out):
    c = jax.lax.axis_index("c")
    s = jax.lax.axis_index("s")
    base = (c * 16 + s) * _TOK_PER_SUB
    cp_in = pltpu.make_async_copy(
        logits_ref.at[:, pl.ds(base, _TOK_PER_SUB)], vin, sem_in)
    cp_in.start()
    cp_in.wait()
    for g in range(_GROUPS):
        sl = pl.ds(g * 16, 16)
        l = [vin[e, sl] for e in range(_E)]
        m1 = l[0]
        for e in range(1, _E):
            m1 = jnp.maximum(m1, l[e])
        wun = []
        den = None
        for e in range(_E):
            cnt = None
            for j in range(_E):
                if j == e:
                    continue
                beat = (l[j] >= l[e]) if j < e else (l[j] > l[e])
                b = jnp.where(beat, 1, 0)
                cnt = b if cnt is None else cnt + b
            we = jnp.where(cnt < 2, jnp.exp(l[e] - m1), 0.0)
            wun.append(we)
            den = we if den is None else den + we
        inv = 1.0 / den
        for e in range(_E):
            vout[e, sl] = wun[e] * inv
    cp_out = pltpu.make_async_copy(
        vout, out_ref.at[:, pl.ds(base, _TOK_PER_SUB)], sem_out)
    cp_out.start()
    cp_out.wait()


def _bodyC(xb_ref, h_ref, wt_ref, wb_ref, wbl_ref, o_ref, wb16_ref, wbl16_ref):
    @pl.when(pl.program_id(0) == 0)
    def _():
        wb16_ref[...] = wb_ref[...].astype(jnp.bfloat16)
        wbl16_ref[...] = (_SCALING * wbl_ref[...]).astype(jnp.bfloat16)

    expand = (jax.lax.broadcasted_iota(jnp.int32, (_E, _RMOE), 1) // _R ==
              jax.lax.broadcasted_iota(jnp.int32, (_E, _RMOE), 0)
              ).astype(jnp.float32)
    wfull = jax.lax.dot_general(
        wt_ref[...], expand, (((0,), (0,)), ((), ())),
        preferred_element_type=jnp.float32)
    hw = (h_ref[...].astype(jnp.float32) * wfull).astype(jnp.bfloat16)
    base = jax.lax.dot_general(
        xb_ref[...], wb16_ref[...], (((1,), (1,)), ((), ())),
        preferred_element_type=jnp.float32)
    lora = jax.lax.dot_general(
        hw, wbl16_ref[...], (((1,), (1,)), ((), ())),
        preferred_element_type=jnp.float32)
    o_ref[...] = base + lora


def kernel(x, W_base, W_gate, W_A, W_B):
    xf = x.reshape(_T, _D_IN)

    logitsT, xb16, h16 = pl.pallas_call(
        _bodyA,
        grid=(_NBLK,),
        in_specs=[
            pl.BlockSpec((_BM, _D_IN), lambda i: (i, 0)),
            pl.BlockSpec((_E, _D_IN), lambda i: (0, 0)),
            pl.BlockSpec((_RMOE, _D_IN), lambda i: (0, 0)),
        ],
        out_specs=[
            pl.BlockSpec((_E, _BM), lambda i: (0, i)),
            pl.BlockSpec((_BM, _D_IN), lambda i: (i, 0)),
            pl.BlockSpec((_BM, _RMOE), lambda i: (i, 0)),
        ],
        out_shape=[
            jax.ShapeDtypeStruct((_E, _T), jnp.float32),
            jax.ShapeDtypeStruct((_T, _D_IN), jnp.bfloat16),
            jax.ShapeDtypeStruct((_T, _RMOE), jnp.bfloat16),
        ],
        scratch_shapes=[pltpu.VMEM((_RMOE, _D_IN), jnp.bfloat16)],
        compiler_params=pltpu.CompilerParams(
            dimension_semantics=("arbitrary",),
            vmem_limit_bytes=100 * 1024 * 1024,
        ),
    )(xf, W_gate, W_A)

    sc_gate = pl.kernel(
        _sc_gate_body,
        out_type=jax.ShapeDtypeStruct((_E, _T), jnp.float32),
        mesh=plsc.VectorSubcoreMesh(core_axis_name="c", subcore_axis_name="s"),
        scratch_types=[
            pltpu.VMEM((_E, _TOK_PER_SUB), jnp.float32),
            pltpu.VMEM((_E, _TOK_PER_SUB), jnp.float32),
            pltpu.SemaphoreType.DMA,
            pltpu.SemaphoreType.DMA,
        ],
    )
    weightsT = sc_gate(logitsT)

    out = pl.pallas_call(
        _bodyC,
        grid=(_NBLK,),
        in_specs=[
            pl.BlockSpec((_BM, _D_IN), lambda i: (i, 0)),
            pl.BlockSpec((_BM, _RMOE), lambda i: (i, 0)),
            pl.BlockSpec((_E, _BM), lambda i: (0, i)),
            pl.BlockSpec((_D_OUT, _D_IN), lambda i: (0, 0)),
            pl.BlockSpec((_D_OUT, _RMOE), lambda i: (0, 0)),
        ],
        out_specs=pl.BlockSpec((_BM, _D_OUT), lambda i: (i, 0)),
        out_shape=jax.ShapeDtypeStruct((_T, _D_OUT), jnp.float32),
        scratch_shapes=[
            pltpu.VMEM((_D_OUT, _D_IN), jnp.bfloat16),
            pltpu.VMEM((_D_OUT, _RMOE), jnp.bfloat16),
        ],
        compiler_params=pltpu.CompilerParams(
            dimension_semantics=("arbitrary",),
            vmem_limit_bytes=100 * 1024 * 1024,
        ),
    )(xb16, h16, weightsT, W_base, W_B)
    return out.reshape(_B, _S, _D_OUT)


# speedup vs baseline: 1.3595x; 1.3595x over previous
import jax
import jax.numpy as jnp
import numpy as np
from jax.experimental import pallas as pl
from jax.experimental.pallas import tpu as pltpu

_B, _S, _D_IN, _D_OUT = 2, 2048, 2048, 2048
_E, _R = 8, 64
_RMOE = _E * _R
_KC = _D_IN + _RMOE
_SCALING = 16.0 / 64.0

_BM = 512


def _body(x_ref, wb_ref, wg_ref, wa_ref, wbl_ref, o_ref,
          wcomb_ref, wa16_ref, xcomb_ref):
    @pl.when(pl.program_id(0) == 0)
    def _cast_weights():
        wcomb_ref[:, :_D_IN] = wb_ref[...].astype(jnp.bfloat16)
        wcomb_ref[:, _D_IN:] = (_SCALING * wbl_ref[...]).astype(jnp.bfloat16)
        wa16_ref[...] = wa_ref[...].astype(jnp.bfloat16)

    xb = x_ref[...].astype(jnp.bfloat16)
    xcomb_ref[:, :_D_IN] = xb

    lT = jax.lax.dot_general(
        wg_ref[...].astype(jnp.bfloat16), xb, (((1,), (1,)), ((), ())),
        preferred_element_type=jnp.float32)
    lj = lT[:, None, :]
    le = lT[None, :, :]
    j_idx = jax.lax.broadcasted_iota(jnp.int32, (_E, _E, _BM), 0)
    e_idx = jax.lax.broadcasted_iota(jnp.int32, (_E, _E, _BM), 1)
    beats = (lj > le) | ((lj == le) & (j_idx < e_idx))
    rank = jnp.sum(beats.astype(jnp.int32), axis=0)
    m1 = jnp.max(lT, axis=0, keepdims=True)
    wun = jnp.where(rank < 2, jnp.exp(lT - m1), 0.0)
    wtsT = wun / jnp.sum(wun, axis=0, keepdims=True)

    expand = (jax.lax.broadcasted_iota(jnp.int32, (_E, _RMOE), 1) // _R ==
              jax.lax.broadcasted_iota(jnp.int32, (_E, _RMOE), 0)
              ).astype(jnp.float32)
    wfull = jax.lax.dot_general(
        wtsT, expand, (((0,), (0,)), ((), ())),
        preferred_element_type=jnp.float32)

    h = jax.lax.dot_general(
        xb, wa16_ref[...], (((1,), (1,)), ((), ())),
        preferred_element_type=jnp.float32)
    xcomb_ref[:, _D_IN:] = (h * wfull).astype(jnp.bfloat16)

    o_ref[...] = jax.lax.dot_general(
        xcomb_ref[...], wcomb_ref[...], (((1,), (1,)), ((), ())),
        preferred_element_type=jnp.float32)


def kernel(x, W_base, W_gate, W_A, W_B):
    xf = x.reshape(_B * _S, _D_IN)

    n_blocks = (_B * _S) // _BM
    out = pl.pallas_call(
        _body,
        grid=(n_blocks,),
        in_specs=[
            pl.BlockSpec((_BM, _D_IN), lambda i: (i, 0)),
            pl.BlockSpec((_D_OUT, _D_IN), lambda i: (0, 0)),
            pl.BlockSpec((_E, _D_IN), lambda i: (0, 0)),
            pl.BlockSpec((_RMOE, _D_IN), lambda i: (0, 0)),
            pl.BlockSpec((_D_OUT, _RMOE), lambda i: (0, 0)),
        ],
        out_specs=pl.BlockSpec((_BM, _D_OUT), lambda i: (i, 0)),
        out_shape=jax.ShapeDtypeStruct((_B * _S, _D_OUT), jnp.float32),
        scratch_shapes=[
            pltpu.VMEM((_D_OUT, _KC), jnp.bfloat16),
            pltpu.VMEM((_RMOE, _D_IN), jnp.bfloat16),
            pltpu.VMEM((_BM, _KC), jnp.bfloat16),
        ],
        compiler_params=pltpu.CompilerParams(
            dimension_semantics=("arbitrary",),
            vmem_limit_bytes=100 * 1024 * 1024,
        ),
    )(xf, W_base, W_gate, W_A, W_B)
    return out.reshape(_B, _S, _D_OUT)
